# baseline (speedup 1.0000x reference)
import functools
import math

import jax
import jax.numpy as jnp
from jax import lax
from jax.experimental import pallas as pl
from jax.experimental.pallas import tpu as pltpu
from jax.experimental.pallas import tpu_sc as plsc

VOCAB = 100000
EMBED = 64
MAX_LEN = 512
BATCH = 1024
SEQ = 200

NC, NS, L = 2, 16, 16
NW = NC * NS
BT = 128
NBT = BATCH // BT
NTASK = SEQ * NBT
TPW = NTASK // NW
NGB = BT // L


def _positional(max_len, d):
    position = jnp.arange(max_len, dtype=jnp.float32)[:, None]
    div = jnp.exp(
        jnp.arange(0, d, 2, dtype=jnp.float32) * (-math.log(10000.0) / d)
    )
    pe = jnp.zeros((max_len, d), jnp.float32)
    pe = pe.at[:, 0::2].set(jnp.sin(position * div))
    pe = pe.at[:, 1::2].set(jnp.cos(position * div))
    return pe


def _body(seq2, table, pe2, out, idx_v, rows_v, out_v, pe_v, sem):
    wid = lax.axis_index("s") * NC + lax.axis_index("c")
    pltpu.sync_copy(pe2, pe_v)
    t0 = wid * TPW

    iotas = [jnp.arange(L, dtype=jnp.int32) + g * L for g in range(NGB)]

    def task(i, _):
        r = t0 + i
        s = r // NBT
        bt = r % NBT
        pltpu.sync_copy(seq2.at[r], idx_v)
        pltpu.async_copy(table.at[idx_v], rows_v, sem).wait()

        ws = []
        for g in range(NGB):
            iv = idx_v[pl.ds(g * L, L)]
            ws.append(
                jnp.where(iv == 0, jnp.float32(0.0), jnp.float32(1.0))
            )

        ps = jnp.full((L,), s // 2, jnp.int32)
        poff = (s % 2) * EMBED

        def ccol(c, _):
            pe_vec = plsc.load_gather(
                pe_v, [ps, jnp.full((L,), poff + c, jnp.int32)]
            )
            cc = jnp.full((L,), c, jnp.int32)
            for g in range(NGB):
                col = plsc.load_gather(rows_v, [iotas[g], cc])
                out_v[c, pl.ds(g * L, L)] = col * ws[g] + pe_vec
            return 0

        lax.fori_loop(0, EMBED, ccol, 0)

        dst = out.at[s].at[:, pl.ds(pl.multiple_of(bt * BT, BT), BT)]
        pltpu.sync_copy(out_v, dst)
        return 0

    lax.fori_loop(0, TPW, task, 0)


@jax.jit
def kernel(sequence, token_table):
    seq2 = (
        sequence.astype(jnp.int32).T.reshape(SEQ * NBT, BT)
    )
    pe2 = _positional(MAX_LEN, EMBED)[:SEQ].reshape(SEQ // 2, 2 * EMBED)

    run = functools.partial(
        pl.kernel,
        out_type=jax.ShapeDtypeStruct((SEQ, EMBED, BATCH), jnp.float32),
        mesh=plsc.VectorSubcoreMesh(core_axis_name="c", subcore_axis_name="s"),
        compiler_params=pltpu.CompilerParams(
            needs_layout_passes=False, use_tc_tiling_on_sc=False
        ),
        scratch_types=[
            pltpu.VMEM((BT,), jnp.int32),
            pltpu.VMEM((BT, EMBED), jnp.float32),
            pltpu.VMEM((EMBED, BT), jnp.float32),
            pltpu.VMEM((SEQ // 2, 2 * EMBED), jnp.float32),
            pltpu.SemaphoreType.DMA,
        ],
    )(_body)
    x = run(seq2, token_table, pe2)
    return jnp.transpose(x, (2, 0, 1))


# speedup vs baseline: 1.8920x; 1.8920x over previous
import functools
import math

import jax
import jax.numpy as jnp
from jax import lax
from jax.experimental import pallas as pl
from jax.experimental.pallas import tpu as pltpu
from jax.experimental.pallas import tpu_sc as plsc

VOCAB = 100000
EMBED = 64
MAX_LEN = 512
BATCH = 1024
SEQ = 200

NC, NS, L = 2, 16, 16
NW = NC * NS
BT = 128
NBT = BATCH // BT
NTASK = SEQ * NBT
TPW = NTASK // NW
NGB = BT // L


def _positional(max_len, d):
    position = jnp.arange(max_len, dtype=jnp.float32)[:, None]
    div = jnp.exp(
        jnp.arange(0, d, 2, dtype=jnp.float32) * (-math.log(10000.0) / d)
    )
    pe = jnp.zeros((max_len, d), jnp.float32)
    pe = pe.at[:, 0::2].set(jnp.sin(position * div))
    pe = pe.at[:, 1::2].set(jnp.cos(position * div))
    return pe


def _body(seq2, table, pe2, out, idx_all, rows0, rows1, outv0, outv1, pe_v,
          g0, g1, s0, s1):
    wid = lax.axis_index("s") * NC + lax.axis_index("c")
    pltpu.sync_copy(pe2, pe_v)
    t0 = wid * TPW
    pltpu.sync_copy(seq2.at[pl.ds(t0, TPW)], idx_all)

    iotas = [jnp.arange(L, dtype=jnp.int32) + g * L for g in range(NGB)]

    def start_gather(i, rows_ref, sem):
        pltpu.async_copy(table.at[idx_all.at[i]], rows_ref, sem)

    def wait_gather(rows_ref, sem):
        pltpu.make_async_copy(table.at[idx_all.at[0]], rows_ref, sem).wait()

    def store_dst(r):
        s = r // NBT
        bt = r % NBT
        return out.at[s].at[:, pl.ds(pl.multiple_of(bt * BT, BT), BT)]

    def start_store(r, outv_ref, sem):
        pltpu.async_copy(outv_ref, store_dst(r), sem)

    def wait_store(outv_ref, sem):
        pltpu.make_async_copy(outv_ref, store_dst(0), sem).wait()

    def compute(i, r, rows_ref, outv_ref):
        s = r // NBT
        ws = []
        for g in range(NGB):
            iv = idx_all[i, pl.ds(g * L, L)]
            ws.append(jnp.where(iv == 0, jnp.float32(0.0), jnp.float32(1.0)))
        ps = jnp.full((L,), s // 2, jnp.int32)
        poff = (s % 2) * EMBED

        @plsc.parallel_loop(0, EMBED, step=1, unroll=2)
        def ccol(c):
            pe_vec = plsc.load_gather(
                pe_v, [ps, jnp.full((L,), poff + c, jnp.int32)]
            )
            cc = jnp.full((L,), c, jnp.int32)
            for g in range(NGB):
                col = plsc.load_gather(rows_ref, [iotas[g], cc])
                outv_ref[c, pl.ds(g * L, L)] = col * ws[g] + pe_vec

    start_gather(0, rows0, g0)

    def step(j, _):
        ia, ib = 2 * j, 2 * j + 1
        ra, rb = t0 + ia, t0 + ib

        start_gather(ib, rows1, g1)

        wait_gather(rows0, g0)

        @pl.when(j > 0)
        def _():
            wait_store(outv0, s0)

        compute(ia, ra, rows0, outv0)
        start_store(ra, outv0, s0)

        @pl.when(j < TPW // 2 - 1)
        def _():
            start_gather(ia + 2, rows0, g0)

        wait_gather(rows1, g1)

        @pl.when(j > 0)
        def _():
            wait_store(outv1, s1)

        compute(ib, rb, rows1, outv1)
        start_store(rb, outv1, s1)
        return 0

    lax.fori_loop(0, TPW // 2, step, 0)
    wait_store(outv0, s0)
    wait_store(outv1, s1)


@jax.jit
def kernel(sequence, token_table):
    seq2 = sequence.astype(jnp.int32).T.reshape(SEQ * NBT, BT)
    pe2 = _positional(MAX_LEN, EMBED)[:SEQ].reshape(SEQ // 2, 2 * EMBED)

    run = functools.partial(
        pl.kernel,
        out_type=jax.ShapeDtypeStruct((SEQ, EMBED, BATCH), jnp.float32),
        mesh=plsc.VectorSubcoreMesh(core_axis_name="c", subcore_axis_name="s"),
        compiler_params=pltpu.CompilerParams(
            needs_layout_passes=False, use_tc_tiling_on_sc=False
        ),
        scratch_types=[
            pltpu.VMEM((TPW, BT), jnp.int32),
            pltpu.VMEM((BT, EMBED), jnp.float32),
            pltpu.VMEM((BT, EMBED), jnp.float32),
            pltpu.VMEM((EMBED, BT), jnp.float32),
            pltpu.VMEM((EMBED, BT), jnp.float32),
            pltpu.VMEM((SEQ // 2, 2 * EMBED), jnp.float32),
            pltpu.SemaphoreType.DMA,
            pltpu.SemaphoreType.DMA,
            pltpu.SemaphoreType.DMA,
            pltpu.SemaphoreType.DMA,
        ],
    )(_body)
    x = run(seq2, token_table, pe2)
    return jnp.transpose(x, (2, 0, 1))


# speedup vs baseline: 1.9096x; 1.0093x over previous
import functools
import math

import jax
import jax.numpy as jnp
from jax import lax
from jax.experimental import pallas as pl
from jax.experimental.pallas import tpu as pltpu
from jax.experimental.pallas import tpu_sc as plsc

VOCAB = 100000
EMBED = 64
MAX_LEN = 512
BATCH = 1024
SEQ = 200

NC, NS, L = 2, 16, 16
NW = NC * NS
BT = 128
NBT = BATCH // BT
NTASK = SEQ * NBT
TPW = NTASK // NW
NGB = BT // L


def _positional(max_len, d):
    position = jnp.arange(max_len, dtype=jnp.float32)[:, None]
    div = jnp.exp(
        jnp.arange(0, d, 2, dtype=jnp.float32) * (-math.log(10000.0) / d)
    )
    pe = jnp.zeros((max_len, d), jnp.float32)
    pe = pe.at[:, 0::2].set(jnp.sin(position * div))
    pe = pe.at[:, 1::2].set(jnp.cos(position * div))
    return pe


def _body(seq2, table, pe2, out, idx_all, rows0, rows1, outv0, outv1, pe_v,
          g0, g1, s0, s1):
    wid = lax.axis_index("s") * NC + lax.axis_index("c")
    pltpu.sync_copy(pe2, pe_v)
    t0 = wid * TPW
    pltpu.sync_copy(seq2.at[pl.ds(t0, TPW)], idx_all)

    iotas = [jnp.arange(L, dtype=jnp.int32) + g * L for g in range(NGB)]

    def start_gather(i, rows_ref, sem):
        pltpu.async_copy(table.at[idx_all.at[i]], rows_ref, sem)

    def wait_gather(rows_ref, sem):
        pltpu.make_async_copy(table.at[idx_all.at[0]], rows_ref, sem).wait()

    def store_dst(r):
        s = r // NBT
        bt = r % NBT
        return out.at[s].at[:, pl.ds(pl.multiple_of(bt * BT, BT), BT)]

    def start_store(r, outv_ref, sem):
        pltpu.async_copy(outv_ref, store_dst(r), sem)

    def wait_store(outv_ref, sem):
        pltpu.make_async_copy(outv_ref, store_dst(0), sem).wait()

    def compute(i, r, rows_ref, outv_ref):
        s = r // NBT
        ps = jnp.full((L,), s // 2, jnp.int32)
        poff = (s % 2) * EMBED
        nz = jnp.int32(0)
        for g in range(NGB):
            iv = idx_all[i, pl.ds(g * L, L)]
            nz = nz + plsc.all_reduce_population_count(iv == 0)[0]

        @pl.when(nz == 0)
        def _fast():
            @plsc.parallel_loop(0, EMBED, step=1, unroll=4)
            def ccol(c):
                pe_vec = plsc.load_gather(
                    pe_v, [ps, jnp.full((L,), poff + c, jnp.int32)]
                )
                cc = jnp.full((L,), c, jnp.int32)
                for g in range(NGB):
                    col = plsc.load_gather(rows_ref, [iotas[g], cc])
                    outv_ref[c, pl.ds(g * L, L)] = col + pe_vec

        @pl.when(nz > 0)
        def _masked():
            ws = []
            for g in range(NGB):
                iv = idx_all[i, pl.ds(g * L, L)]
                ws.append(
                    jnp.where(iv == 0, jnp.float32(0.0), jnp.float32(1.0))
                )

            @plsc.parallel_loop(0, EMBED, step=1, unroll=2)
            def ccol(c):
                pe_vec = plsc.load_gather(
                    pe_v, [ps, jnp.full((L,), poff + c, jnp.int32)]
                )
                cc = jnp.full((L,), c, jnp.int32)
                for g in range(NGB):
                    col = plsc.load_gather(rows_ref, [iotas[g], cc])
                    outv_ref[c, pl.ds(g * L, L)] = col * ws[g] + pe_vec

    start_gather(0, rows0, g0)

    def step(j, _):
        ia, ib = 2 * j, 2 * j + 1
        ra, rb = t0 + ia, t0 + ib

        start_gather(ib, rows1, g1)

        wait_gather(rows0, g0)

        @pl.when(j > 0)
        def _():
            wait_store(outv0, s0)

        compute(ia, ra, rows0, outv0)
        start_store(ra, outv0, s0)

        @pl.when(j < TPW // 2 - 1)
        def _():
            start_gather(ia + 2, rows0, g0)

        wait_gather(rows1, g1)

        @pl.when(j > 0)
        def _():
            wait_store(outv1, s1)

        compute(ib, rb, rows1, outv1)
        start_store(rb, outv1, s1)
        return 0

    lax.fori_loop(0, TPW // 2, step, 0)
    wait_store(outv0, s0)
    wait_store(outv1, s1)


@jax.jit
def kernel(sequence, token_table):
    seq2 = sequence.astype(jnp.int32).T.reshape(SEQ * NBT, BT)
    pe2 = _positional(MAX_LEN, EMBED)[:SEQ].reshape(SEQ // 2, 2 * EMBED)

    run = functools.partial(
        pl.kernel,
        out_type=jax.ShapeDtypeStruct((SEQ, EMBED, BATCH), jnp.float32),
        mesh=plsc.VectorSubcoreMesh(core_axis_name="c", subcore_axis_name="s"),
        compiler_params=pltpu.CompilerParams(
            needs_layout_passes=False, use_tc_tiling_on_sc=False
        ),
        scratch_types=[
            pltpu.VMEM((TPW, BT), jnp.int32),
            pltpu.VMEM((BT, EMBED), jnp.float32),
            pltpu.VMEM((BT, EMBED), jnp.float32),
            pltpu.VMEM((EMBED, BT), jnp.float32),
            pltpu.VMEM((EMBED, BT), jnp.float32),
            pltpu.VMEM((SEQ // 2, 2 * EMBED), jnp.float32),
            pltpu.SemaphoreType.DMA,
            pltpu.SemaphoreType.DMA,
            pltpu.SemaphoreType.DMA,
            pltpu.SemaphoreType.DMA,
        ],
    )(_body)
    x = run(seq2, token_table, pe2)
    return jnp.transpose(x, (2, 0, 1))


# speedup vs baseline: 2.2689x; 1.1882x over previous
import functools
import math

import jax
import jax.numpy as jnp
from jax import lax
from jax.experimental import pallas as pl
from jax.experimental.pallas import tpu as pltpu
from jax.experimental.pallas import tpu_sc as plsc

VOCAB = 100000
EMBED = 64
MAX_LEN = 512
BATCH = 1024
SEQ = 200

NC, NS, L = 2, 16, 16
NW = NC * NS
BT = 128
NBT = BATCH // BT
NTASK = SEQ * NBT
TPW = NTASK // NW
NGB = BT // L


def _positional(max_len, d):
    position = jnp.arange(max_len, dtype=jnp.float32)[:, None]
    div = jnp.exp(
        jnp.arange(0, d, 2, dtype=jnp.float32) * (-math.log(10000.0) / d)
    )
    pe = jnp.zeros((max_len, d), jnp.float32)
    pe = pe.at[:, 0::2].set(jnp.sin(position * div))
    pe = pe.at[:, 1::2].set(jnp.cos(position * div))
    return pe


def _body(seq2, table, pe2, out, idx_all, rows0, rows1, outv0, outv1, pe_v,
          g0, g1, s0, s1):
    wid = lax.axis_index("s") * NC + lax.axis_index("c")
    pltpu.sync_copy(pe2, pe_v)
    t0 = wid * TPW
    pltpu.sync_copy(seq2.at[pl.ds(t0, TPW)], idx_all)

    iotas = [jnp.arange(L, dtype=jnp.int32) + g * L for g in range(NGB)]

    def start_gather(i, rows_ref, sem):
        pltpu.async_copy(table.at[idx_all.at[i]], rows_ref, sem)

    def wait_gather(rows_ref, sem):
        pltpu.make_async_copy(table.at[idx_all.at[0]], rows_ref, sem).wait()

    def task_sbt(r):
        return (r // 64) * 8 + r % 8, (r % 64) // 8

    def store_dst(r):
        s, bt = task_sbt(r)
        return out.at[s].at[:, bt]

    def start_store(r, outv_ref, sem):
        pltpu.async_copy(outv_ref, store_dst(r), sem)

    def wait_store(outv_ref, sem):
        pltpu.make_async_copy(outv_ref, store_dst(0), sem).wait()

    def compute(i, r, rows_ref, outv_ref):
        s, _ = task_sbt(r)
        ps = jnp.full((L,), s // 2, jnp.int32)
        poff = (s % 2) * EMBED
        nz = jnp.int32(0)
        for g in range(NGB):
            iv = idx_all[i, pl.ds(g * L, L)]
            nz = nz + plsc.all_reduce_population_count(iv == 0)[0]

        @pl.when(nz == 0)
        def _fast():
            @plsc.parallel_loop(0, EMBED, step=1, unroll=4)
            def ccol(c):
                pe_vec = plsc.load_gather(
                    pe_v, [ps, jnp.full((L,), poff + c, jnp.int32)]
                )
                cc = jnp.full((L,), c, jnp.int32)
                for g in range(NGB):
                    col = plsc.load_gather(rows_ref, [iotas[g], cc])
                    outv_ref[c // 8, c % 8, pl.ds(g * L, L)] = col + pe_vec

        @pl.when(nz > 0)
        def _masked():
            ws = []
            for g in range(NGB):
                iv = idx_all[i, pl.ds(g * L, L)]
                ws.append(
                    jnp.where(iv == 0, jnp.float32(0.0), jnp.float32(1.0))
                )

            @plsc.parallel_loop(0, EMBED, step=1, unroll=2)
            def ccol(c):
                pe_vec = plsc.load_gather(
                    pe_v, [ps, jnp.full((L,), poff + c, jnp.int32)]
                )
                cc = jnp.full((L,), c, jnp.int32)
                for g in range(NGB):
                    col = plsc.load_gather(rows_ref, [iotas[g], cc])
                    outv_ref[c // 8, c % 8, pl.ds(g * L, L)] = (
                        col * ws[g] + pe_vec
                    )

    start_gather(0, rows0, g0)

    def step(j, _):
        ia, ib = 2 * j, 2 * j + 1
        ra, rb = t0 + ia, t0 + ib

        start_gather(ib, rows1, g1)

        wait_gather(rows0, g0)

        @pl.when(j > 0)
        def _():
            wait_store(outv0, s0)

        compute(ia, ra, rows0, outv0)
        start_store(ra, outv0, s0)

        @pl.when(j < TPW // 2 - 1)
        def _():
            start_gather(ia + 2, rows0, g0)

        wait_gather(rows1, g1)

        @pl.when(j > 0)
        def _():
            wait_store(outv1, s1)

        compute(ib, rb, rows1, outv1)
        start_store(rb, outv1, s1)
        return 0

    lax.fori_loop(0, TPW // 2, step, 0)
    wait_store(outv0, s0)
    wait_store(outv1, s1)


@jax.jit
def kernel(sequence, token_table):
    seq2 = (
        sequence.astype(jnp.int32)
        .reshape(NBT, BT, SEQ // 8, 8)
        .transpose(2, 0, 3, 1)
        .reshape(NTASK, BT)
    )
    pe2 = _positional(MAX_LEN, EMBED)[:SEQ].reshape(SEQ // 2, 2 * EMBED)

    run = functools.partial(
        pl.kernel,
        out_type=jax.ShapeDtypeStruct(
            (SEQ, EMBED // 8, NBT, 8, BT), jnp.float32
        ),
        mesh=plsc.VectorSubcoreMesh(core_axis_name="c", subcore_axis_name="s"),
        compiler_params=pltpu.CompilerParams(
            needs_layout_passes=False, use_tc_tiling_on_sc=False
        ),
        scratch_types=[
            pltpu.VMEM((TPW, BT), jnp.int32),
            pltpu.VMEM((BT, EMBED), jnp.float32),
            pltpu.VMEM((BT, EMBED), jnp.float32),
            pltpu.VMEM((EMBED // 8, 8, BT), jnp.float32),
            pltpu.VMEM((EMBED // 8, 8, BT), jnp.float32),
            pltpu.VMEM((SEQ // 2, 2 * EMBED), jnp.float32),
            pltpu.SemaphoreType.DMA,
            pltpu.SemaphoreType.DMA,
            pltpu.SemaphoreType.DMA,
            pltpu.SemaphoreType.DMA,
        ],
    )(_body)
    x = run(seq2, token_table, pe2)
    return (
        x.transpose(2, 4, 0, 1, 3).reshape(BATCH, SEQ, EMBED)
    )
